# Initial kernel scaffold; baseline (speedup 1.0000x reference)
import jax
import jax.numpy as jnp
from jax.experimental import pallas as pl


def kernel(raw_y_hat, t, e, cur_logit_scale):
    raise NotImplementedError("write your pallas kernel here")


# speedup vs baseline: 7.0860x; 7.0860x over previous
import functools

import jax
import jax.numpy as jnp
from jax import lax
from jax.experimental import pallas as pl
from jax.experimental.pallas import tpu as pltpu
from jax.experimental.pallas import tpu_sc as plsc

_B = 4096
_NBINS = 32
_L = 16
_NC = 2
_NS = 16
_NW = _NC * _NS
_ROWS = _B // _NW
_NEG_INF = float("-inf")


def _sc_stats_body(y_hbm, t_hbm, e_hbm, out_hbm, y_v, t_v, e_v, stats_v):
    wid = lax.axis_index("s") * _NC + lax.axis_index("c")
    base = wid * _ROWS
    pltpu.sync_copy(y_hbm.at[pl.ds(base, _ROWS), :], y_v)
    pltpu.sync_copy(t_hbm.at[pl.ds(base, _ROWS)], t_v)
    pltpu.sync_copy(e_hbm.at[pl.ds(base, _ROWS)], e_v)

    bins_lo = lax.iota(jnp.int32, _L)
    bins_hi = bins_lo + _L
    zero = jnp.zeros((_L,), jnp.float32)
    ninf = jnp.full((_L,), _NEG_INF, jnp.float32)

    one = jnp.ones((_L,), jnp.float32)

    def pass1(g, carry):
        mx_lo, mx_hi, p_lo, p_hi, t_lo, t_hi, c_lo, c_hi = carry
        gbase = g * _L
        tv = t_v[pl.ds(gbase, _L)]
        ev = e_v[pl.ds(gbase, _L)]
        for j in range(_L):
            tb = tv[j]
            is_e = ev[j] == 1
            tb_eff = jnp.where(is_e, jnp.int32(_NBINS), tb)
            tb_pos = jnp.where(is_e, tb, jnp.int32(-1))
            y_lo = y_v[gbase + j, pl.ds(0, _L)]
            y_hi = y_v[gbase + j, pl.ds(_L, _L)]
            m_lo = bins_lo < tb_eff
            m_hi = bins_hi < tb_eff
            pos_lo = bins_lo == tb_pos
            pos_hi = bins_hi == tb_pos
            mx_lo = jnp.maximum(mx_lo, jnp.where(m_lo, y_lo, ninf))
            mx_hi = jnp.maximum(mx_hi, jnp.where(m_hi, y_hi, ninf))
            p_lo = p_lo + jnp.where(pos_lo, y_lo, zero)
            p_hi = p_hi + jnp.where(pos_hi, y_hi, zero)
            t_lo = t_lo + jnp.where(pos_lo, one, zero)
            t_hi = t_hi + jnp.where(pos_hi, one, zero)
            c_lo = c_lo + jnp.where(m_lo, one, zero)
            c_hi = c_hi + jnp.where(m_hi, one, zero)
        return (mx_lo, mx_hi, p_lo, p_hi, t_lo, t_hi, c_lo, c_hi)

    init1 = (ninf, ninf, zero, zero, zero, zero, zero, zero)
    mx_lo, mx_hi, p_lo, p_hi, t_lo, t_hi, c_lo, c_hi = lax.fori_loop(
        0, _ROWS // _L, pass1, init1)

    def pass2(g, carry):
        s_lo, s_hi = carry
        gbase = g * _L
        tv = t_v[pl.ds(gbase, _L)]
        ev = e_v[pl.ds(gbase, _L)]
        for j in range(_L):
            tb = tv[j]
            is_e = ev[j] == 1
            tb_eff = jnp.where(is_e, jnp.int32(_NBINS), tb)
            y_lo = y_v[gbase + j, pl.ds(0, _L)]
            y_hi = y_v[gbase + j, pl.ds(_L, _L)]
            m_lo = bins_lo < tb_eff
            m_hi = bins_hi < tb_eff
            e_lo = jnp.exp(jnp.where(m_lo, y_lo - mx_lo, zero))
            e_hi = jnp.exp(jnp.where(m_hi, y_hi - mx_hi, zero))
            s_lo = s_lo + jnp.where(m_lo, e_lo, zero)
            s_hi = s_hi + jnp.where(m_hi, e_hi, zero)
        return (s_lo, s_hi)

    s_lo, s_hi = lax.fori_loop(0, _ROWS // _L, pass2, (zero, zero))

    stats_v[0, :] = mx_lo
    stats_v[1, :] = mx_hi
    stats_v[2, :] = s_lo
    stats_v[3, :] = s_hi
    stats_v[4, :] = p_lo
    stats_v[5, :] = p_hi
    stats_v[6, :] = t_lo
    stats_v[7, :] = t_hi
    stats_v[8, :] = c_lo
    stats_v[9, :] = c_hi
    pltpu.sync_copy(stats_v, out_hbm.at[wid])


@functools.partial(jax.jit, static_argnames=())
def _sc_stats(y, t, e):
    mesh = plsc.VectorSubcoreMesh(core_axis_name="c", subcore_axis_name="s")
    fn = functools.partial(
        pl.kernel,
        mesh=mesh,
        out_type=jax.ShapeDtypeStruct((_NW, 10, _L), jnp.float32),
        scratch_types=[
            pltpu.VMEM((_ROWS, _NBINS), jnp.float32),
            pltpu.VMEM((_ROWS,), jnp.int32),
            pltpu.VMEM((_ROWS,), jnp.int32),
            pltpu.VMEM((10, _L), jnp.float32),
        ],
    )(_sc_stats_body)
    return fn(y, t, e)


def _tc_finish_body(mx_ref, s_ref, p_ref, t_ref, c_ref, scale_ref, out_ref):
    mx = mx_ref[:]
    m = jnp.max(mx, axis=0, keepdims=True)
    m_safe = jnp.where(jnp.isfinite(m), m, 0.0)
    s = jnp.sum(s_ref[:] * jnp.exp(mx - m_safe), axis=0, keepdims=True)
    p = jnp.sum(p_ref[:], axis=0, keepdims=True)
    t_sum = jnp.sum(t_ref[:], axis=0, keepdims=True)
    cnt = jnp.sum(c_ref[:], axis=0, keepdims=True)
    loss = m_safe + jnp.log(s) - p / t_sum
    valid = (cnt > 0.0) & (t_sum > 0.0)
    total = jnp.sum(jnp.where(valid, loss, 0.0))
    num = jnp.sum(jnp.where(valid, 1.0, 0.0))
    denom = jnp.where(num != 0.0, num, 1.0)
    val = total / denom + 0.0 * scale_ref[0, 0]
    out_ref[:, :] = jnp.full((1, 1), val, jnp.float32)


def _tc_finish(mx, s, p, t_sum, cnt, scale):
    return pl.pallas_call(
        _tc_finish_body,
        out_shape=jax.ShapeDtypeStruct((1, 1), jnp.float32),
    )(mx, s, p, t_sum, cnt, scale)


def kernel(raw_y_hat, t, e, cur_logit_scale):
    y = raw_y_hat.astype(jnp.float32)
    stats = _sc_stats(y, t.astype(jnp.int32), e.astype(jnp.int32))
    stats = stats.reshape(_NW, 5, _NBINS)
    mx = stats[:, 0, :]
    s = stats[:, 1, :]
    p = stats[:, 2, :]
    t_sum = stats[:, 3, :]
    cnt = stats[:, 4, :]
    scale = jnp.asarray(cur_logit_scale, jnp.float32).reshape(1, 1)
    out = _tc_finish(mx, s, p, t_sum, cnt, scale)
    return out.reshape(()).astype(raw_y_hat.dtype)


# speedup vs baseline: 8.2362x; 1.1623x over previous
import functools

import jax
import jax.numpy as jnp
from jax import lax
from jax.experimental import pallas as pl
from jax.experimental.pallas import tpu as pltpu
from jax.experimental.pallas import tpu_sc as plsc

_B = 4096
_NBINS = 32
_L = 16
_NC = 2
_NS = 16
_NW = _NC * _NS
_ROWS = _B // _NW
_NEG_INF = float("-inf")


def _sc_stats_body(y_hbm, t_hbm, e_hbm, out_hbm, y_v, t_v, e_v, stats_v):
    wid = lax.axis_index("s") * _NC + lax.axis_index("c")
    base = wid * _ROWS
    pltpu.sync_copy(y_hbm.at[pl.ds(base, _ROWS), :], y_v)
    pltpu.sync_copy(t_hbm.at[pl.ds(base, _ROWS)], t_v)
    pltpu.sync_copy(e_hbm.at[pl.ds(base, _ROWS)], e_v)

    bins_lo = lax.iota(jnp.int32, _L)
    bins_hi = bins_lo + _L
    zero = jnp.zeros((_L,), jnp.float32)
    ninf = jnp.full((_L,), _NEG_INF, jnp.float32)

    one = jnp.ones((_L,), jnp.float32)

    def pass1(g, carry):
        mx_lo, mx_hi, p_lo, p_hi, t_lo, t_hi, c_lo, c_hi = carry
        gbase = g * _L
        tv = t_v[pl.ds(gbase, _L)]
        ev = e_v[pl.ds(gbase, _L)]
        for j in range(_L):
            tb = tv[j]
            is_e = ev[j] == 1
            tb_eff = jnp.where(is_e, jnp.int32(_NBINS), tb)
            tb_pos = jnp.where(is_e, tb, jnp.int32(-1))
            y_lo = y_v[gbase + j, pl.ds(0, _L)]
            y_hi = y_v[gbase + j, pl.ds(_L, _L)]
            m_lo = bins_lo < tb_eff
            m_hi = bins_hi < tb_eff
            pos_lo = bins_lo == tb_pos
            pos_hi = bins_hi == tb_pos
            mx_lo = jnp.maximum(mx_lo, jnp.where(m_lo, y_lo, ninf))
            mx_hi = jnp.maximum(mx_hi, jnp.where(m_hi, y_hi, ninf))
            p_lo = p_lo + jnp.where(pos_lo, y_lo, zero)
            p_hi = p_hi + jnp.where(pos_hi, y_hi, zero)
            t_lo = t_lo + jnp.where(pos_lo, one, zero)
            t_hi = t_hi + jnp.where(pos_hi, one, zero)
            c_lo = c_lo + jnp.where(m_lo, one, zero)
            c_hi = c_hi + jnp.where(m_hi, one, zero)
        return (mx_lo, mx_hi, p_lo, p_hi, t_lo, t_hi, c_lo, c_hi)

    init1 = (ninf, ninf, zero, zero, zero, zero, zero, zero)
    mx_lo, mx_hi, p_lo, p_hi, t_lo, t_hi, c_lo, c_hi = lax.fori_loop(
        0, _ROWS // _L, pass1, init1)

    def pass2(g, carry):
        s_lo, s_hi = carry
        gbase = g * _L
        tv = t_v[pl.ds(gbase, _L)]
        ev = e_v[pl.ds(gbase, _L)]
        for j in range(_L):
            tb = tv[j]
            is_e = ev[j] == 1
            tb_eff = jnp.where(is_e, jnp.int32(_NBINS), tb)
            y_lo = y_v[gbase + j, pl.ds(0, _L)]
            y_hi = y_v[gbase + j, pl.ds(_L, _L)]
            m_lo = bins_lo < tb_eff
            m_hi = bins_hi < tb_eff
            e_lo = jnp.exp(jnp.where(m_lo, y_lo - mx_lo, zero))
            e_hi = jnp.exp(jnp.where(m_hi, y_hi - mx_hi, zero))
            s_lo = s_lo + jnp.where(m_lo, e_lo, zero)
            s_hi = s_hi + jnp.where(m_hi, e_hi, zero)
        return (s_lo, s_hi)

    s_lo, s_hi = lax.fori_loop(0, _ROWS // _L, pass2, (zero, zero))

    stats_v[0, :] = mx_lo
    stats_v[1, :] = mx_hi
    stats_v[2, :] = s_lo
    stats_v[3, :] = s_hi
    stats_v[4, :] = p_lo
    stats_v[5, :] = p_hi
    stats_v[6, :] = t_lo
    stats_v[7, :] = t_hi
    stats_v[8, :] = c_lo
    stats_v[9, :] = c_hi
    pltpu.sync_copy(stats_v, out_hbm.at[wid])


@functools.partial(jax.jit, static_argnames=())
def _sc_stats(y, t, e):
    mesh = plsc.VectorSubcoreMesh(core_axis_name="c", subcore_axis_name="s")
    fn = functools.partial(
        pl.kernel,
        mesh=mesh,
        out_type=jax.ShapeDtypeStruct((_NW, 10, _L), jnp.float32),
        scratch_types=[
            pltpu.VMEM((_ROWS, _NBINS), jnp.float32),
            pltpu.VMEM((_ROWS,), jnp.int32),
            pltpu.VMEM((_ROWS,), jnp.int32),
            pltpu.VMEM((10, _L), jnp.float32),
        ],
    )(_sc_stats_body)
    return fn(y, t, e)


def _tc_finish_body(stats_ref, scale_ref, out_ref):
    total = jnp.float32(0.0)
    num = jnp.float32(0.0)
    for h in range(2):
        mx = stats_ref[:, 0 + h, :]
        sw = stats_ref[:, 2 + h, :]
        pw = stats_ref[:, 4 + h, :]
        tw = stats_ref[:, 6 + h, :]
        cw = stats_ref[:, 8 + h, :]
        m = jnp.max(mx, axis=0, keepdims=True)
        m_safe = jnp.where(jnp.isfinite(m), m, 0.0)
        s = jnp.sum(sw * jnp.exp(mx - m_safe), axis=0, keepdims=True)
        p = jnp.sum(pw, axis=0, keepdims=True)
        t_sum = jnp.sum(tw, axis=0, keepdims=True)
        cnt = jnp.sum(cw, axis=0, keepdims=True)
        loss = m_safe + jnp.log(s) - p / t_sum
        valid = (cnt > 0.0) & (t_sum > 0.0)
        total = total + jnp.sum(jnp.where(valid, loss, 0.0))
        num = num + jnp.sum(jnp.where(valid, 1.0, 0.0))
    denom = jnp.where(num != 0.0, num, 1.0)
    val = total / denom + 0.0 * scale_ref[0, 0]
    out_ref[:, :] = jnp.full((1, 1), val, jnp.float32)


def _tc_finish(stats, scale):
    return pl.pallas_call(
        _tc_finish_body,
        out_shape=jax.ShapeDtypeStruct((1, 1), jnp.float32),
    )(stats, scale)


def kernel(raw_y_hat, t, e, cur_logit_scale):
    y = raw_y_hat.astype(jnp.float32)
    stats = _sc_stats(y, t.astype(jnp.int32), e.astype(jnp.int32))
    scale = jnp.asarray(cur_logit_scale, jnp.float32).reshape(1, 1)
    out = _tc_finish(stats, scale)
    return out.reshape(()).astype(raw_y_hat.dtype)
